# Initial kernel scaffold; baseline (speedup 1.0000x reference)
import jax
import jax.numpy as jnp
from jax.experimental import pallas as pl


def kernel(xy_pos, time, skeletons, edge_index, batch, time_emb, enc_W1, enc_b1, enc_W2, enc_b2, pos_W1, pos_b1, pos_W2, pos_b2, mlp_W1, mlp_b1, mlp_W2, mlp_b2, gnn_Wq, gnn_Wk, gnn_Wv, gnn_Wo, gnn_bo, fin_W1, fin_b1, fin_W2, fin_b2):
    raise NotImplementedError("write your pallas kernel here")


# speedup vs baseline: 9.0583x; 9.0583x over previous
import functools

import jax
import jax.numpy as jnp
import numpy as np
from jax import lax
from jax.experimental import pallas as pl
from jax.experimental.pallas import tpu as pltpu
from jax.experimental.pallas import tpu_sc as plsc

N = 10000
E = 160000
T = 10
B = 1000
SKD = 75
D = 320
H = 8
DH = 32

NC = 2
NS = 16
NW = NC * NS
CH = 128

NT_CH = 3
NT = NW * NT_CH * CH
E_CH = 40
EP = NW * E_CH * CH
EPW = E_CH * CH
ACC_PW = 632
ACC_N = NS * ACC_PW
WE_W = 128
GROUPS = ((0, 1, 2), (3, 4, 5), (6, 7))

def _gelu(x):
    return 0.5 * x * (1.0 + lax.erf(x * np.float32(1.0 / np.sqrt(2.0))))


@functools.cache
def _sc_kernels():
    mesh = plsc.VectorSubcoreMesh(core_axis_name="c", subcore_axis_name="s",
                                  num_cores=NC, num_subcores=NS)

    @functools.partial(
        pl.kernel,
        out_type=jax.ShapeDtypeStruct((NT, 128), jnp.float32),
        mesh=mesh,
        scratch_types=[
            pltpu.VMEM((NT_CH, CH), jnp.int32),
            pltpu.VMEM((NT_CH * CH, 128), jnp.float32),
            pltpu.SemaphoreType.DMA,
        ],
    )
    def time_gather(time_hbm, temb_hbm, out_hbm, idx_v, rows_v, sem):
        wid = lax.axis_index("s") * NC + lax.axis_index("c")
        pltpu.sync_copy(time_hbm.at[wid], idx_v)

        def body(j, carry):
            pltpu.async_copy(temb_hbm.at[idx_v.at[j]],
                             rows_v.at[pl.ds(j * CH, CH)], sem).wait()
            return carry

        lax.fori_loop(0, NT_CH, body, 0)
        pltpu.sync_copy(rows_v,
                        out_hbm.at[pl.ds(wid * (NT_CH * CH), NT_CH * CH)])

    @functools.partial(
        pl.kernel,
        out_type=(jax.ShapeDtypeStruct((EP, 256), jnp.float32),
                  jax.ShapeDtypeStruct((EP, 512), jnp.float32)),
        mesh=mesh,
        scratch_types=[
            pltpu.VMEM((E_CH, CH), jnp.int32),
            pltpu.VMEM((E_CH, CH), jnp.int32),
            pltpu.VMEM((CH, 256), jnp.float32),
            pltpu.VMEM((CH, 512), jnp.float32),
            pltpu.SemaphoreType.DMA,
            pltpu.SemaphoreType.DMA,
        ],
    )
    def edge_gather(dst_hbm, src_hbm, q_hbm, kv_hbm, qe_hbm, kve_hbm,
                    idxd, idxs, qrow, kvrow, sem1, sem2):
        wid = lax.axis_index("s") * NC + lax.axis_index("c")
        pltpu.sync_copy(dst_hbm.at[wid], idxd)
        pltpu.sync_copy(src_hbm.at[wid], idxs)

        def body(j, carry):
            base = wid * EPW + j * CH
            cp1 = pltpu.async_copy(q_hbm.at[idxd.at[j]], qrow, sem1)
            cp2 = pltpu.async_copy(kv_hbm.at[idxs.at[j]], kvrow, sem2)
            cp1.wait()
            cp2.wait()
            pltpu.sync_copy(qrow, qe_hbm.at[pl.ds(base, CH)])
            pltpu.sync_copy(kvrow, kve_hbm.at[pl.ds(base, CH)])
            return carry

        lax.fori_loop(0, E_CH, body, 0)

    @functools.partial(
        pl.kernel,
        out_type=jax.ShapeDtypeStruct((NC, ACC_N, WE_W), jnp.float32),
        mesh=mesh,
        scratch_types=[
            pltpu.VMEM((E_CH, CH), jnp.int32),
            pltpu.VMEM((CH, WE_W), jnp.float32),
            pltpu.VMEM_SHARED((ACC_N, WE_W), jnp.float32),
        ],
    )
    def edge_scatter(we_hbm, dst_hbm, zero_hbm, out_hbm, idxd, rows_v, acc):
        cid = lax.axis_index("c")
        sid = lax.axis_index("s")
        wid = sid * NC + cid
        pltpu.sync_copy(zero_hbm, acc.at[pl.ds(sid * ACC_PW, ACC_PW)])
        plsc.subcore_barrier()
        pltpu.sync_copy(dst_hbm.at[wid], idxd)

        def body(j, carry):
            base = wid * EPW + j * CH
            pltpu.sync_copy(we_hbm.at[pl.ds(base, CH)], rows_v)
            pltpu.sync_copy(rows_v, acc.at[idxd.at[j]], add=True)
            return carry

        lax.fori_loop(0, E_CH, body, 0)
        plsc.subcore_barrier()
        pltpu.sync_copy(acc.at[pl.ds(sid * ACC_PW, ACC_PW)],
                        out_hbm.at[cid, pl.ds(sid * ACC_PW, ACC_PW)])

    return time_gather, edge_gather, edge_scatter


def _dense_pre_body(skel_ref, xy_ref, tf_ref, eW1, eb1, eW2, eb2,
                    pW1, pb1, pW2, pb2, mW1, mb1, mW2, mb2,
                    wq, wk, wv, comb_o, q_o, kv_o, kn_o):
    x = skel_ref[...]
    skel = _gelu(x @ eW1[...] + eb1[...]) @ eW2[...] + eb2[...]
    pf = _gelu(xy_ref[...] @ pW1[...] + pb1[...]) @ pW2[...] + pb2[...]
    cin = jnp.concatenate([skel, pf, tf_ref[...]], axis=1)
    comb = _gelu(cin @ mW1[...] + mb1[...]) @ mW2[...] + mb2[...]
    comb_o[...] = comb
    q = comb @ wq[...]
    k = comb @ wk[...]
    v = comb @ wv[...]
    q_o[...] = q
    kv_o[...] = jnp.concatenate([k, v], axis=1)
    ksq = k * k
    kn_o[...] = jnp.sqrt(jnp.concatenate(
        [jnp.sum(ksq[:, h * DH:(h + 1) * DH], axis=1, keepdims=True)
         for h in range(H)], axis=1))


def _edge_dense_body(qe_ref, kve_ref, km_ref, we0_o, we1_o, we2_o):
    qe = qe_ref[...]
    kve = kve_ref[...]
    inv = np.float32(1.0 / np.sqrt(DH))
    per_head = []
    for h in range(H):
        qh = qe[:, h * DH:(h + 1) * DH]
        kh = kve[:, h * DH:(h + 1) * DH]
        vh = kve[:, 256 + h * DH:256 + (h + 1) * DH]
        s = jnp.sum(qh * kh, axis=1, keepdims=True) * inv
        qn = jnp.sqrt(jnp.sum(qh * qh, axis=1, keepdims=True))
        ce = qn * km_ref[0:1, h:h + 1] * inv
        ex = jnp.exp(s - ce)
        per_head.append((ex * vh, ex))
    for hs, o_ref in zip(GROUPS, (we0_o, we1_o, we2_o)):
        wvs = [per_head[h][0] for h in hs]
        exs = [per_head[h][1] for h in hs]
        pad = WE_W - (DH + 1) * len(hs)
        z = jnp.zeros((qe.shape[0], pad), jnp.float32)
        o_ref[...] = jnp.concatenate(wvs + exs + [z], axis=1)


def _final_body(s0a, s0b, s1a, s1b, s2a, s2b, comb_ref, wo, bo,
                fW1, fb1, fW2, fb2, out_o):
    pieces = []
    for hs, (ga, gb) in zip(GROUPS, ((s0a, s0b), (s1a, s1b), (s2a, s2b))):
        g = ga[...] + gb[...]
        nh = len(hs)
        for i in range(nh):
            num = g[:, i * DH:(i + 1) * DH]
            den = g[:, nh * DH + i:nh * DH + i + 1]
            pieces.append(num / (den + 1e-16))
    agg = jnp.concatenate(pieces, axis=1)
    feats = agg @ wo[...] + bo[...]
    x = feats + comb_ref[...]
    hid = _gelu(x @ fW1[...] + fb1[...])
    out_o[...] = hid @ fW2[...] + fb2[...]


def _full(i, o):
    return pl.BlockSpec(o, lambda i_: tuple(0 for _ in o))


def kernel(xy_pos, time, skeletons, edge_index, batch, time_emb,
           enc_W1, enc_b1, enc_W2, enc_b2, pos_W1, pos_b1, pos_W2, pos_b2,
           mlp_W1, mlp_b1, mlp_W2, mlp_b2,
           gnn_Wq, gnn_Wk, gnn_Wv, gnn_Wo, gnn_bo,
           fin_W1, fin_b1, fin_W2, fin_b2):
    _time_gather, _edge_gather, _edge_scatter = _sc_kernels()
    f32 = jnp.float32
    time_i = time.astype(jnp.int32)
    src = edge_index[0].astype(jnp.int32)
    dst = edge_index[1].astype(jnp.int32)

    skel_in = jnp.transpose(skeletons, (1, 0, 2)).reshape(N, SKD)

    time_pad = jnp.concatenate(
        [time_i, jnp.zeros((NT - N,), jnp.int32)]).reshape(NW, NT_CH, CH)
    temb_pad = jnp.pad(time_emb, ((0, 0), (0, 128 - time_emb.shape[1])))
    tfeat = _time_gather(time_pad, temb_pad)[:N, :32]

    BN = 1000
    gn = N // BN
    row = lambda w: pl.BlockSpec((BN, w), lambda i: (i, 0))
    comb, q, kv, kn = pl.pallas_call(
        _dense_pre_body,
        grid=(gn,),
        in_specs=[
            row(SKD), row(1), row(32),
            _full(0, (SKD, 256)), _full(0, (1, 256)),
            _full(0, (256, 256)), _full(0, (1, 256)),
            _full(0, (1, 16)), _full(0, (1, 16)),
            _full(0, (16, 32)), _full(0, (1, 32)),
            _full(0, (D, 128)), _full(0, (1, 128)),
            _full(0, (128, D)), _full(0, (1, D)),
            _full(0, (D, 256)), _full(0, (D, 256)), _full(0, (D, 256)),
        ],
        out_specs=[row(D), row(256), row(512), row(H)],
        out_shape=[
            jax.ShapeDtypeStruct((N, D), f32),
            jax.ShapeDtypeStruct((N, 256), f32),
            jax.ShapeDtypeStruct((N, 512), f32),
            jax.ShapeDtypeStruct((N, H), f32),
        ],
    )(skel_in, xy_pos, tfeat,
      enc_W1, enc_b1.reshape(1, -1), enc_W2, enc_b2.reshape(1, -1),
      pos_W1, pos_b1.reshape(1, -1), pos_W2, pos_b2.reshape(1, -1),
      mlp_W1, mlp_b1.reshape(1, -1), mlp_W2, mlp_b2.reshape(1, -1),
      gnn_Wq, gnn_Wk, gnn_Wv)

    khmax = jnp.max(kn, axis=0).reshape(1, H)

    pad_e = EP - E
    dst_pad = jnp.concatenate([dst, jnp.full((pad_e,), N, jnp.int32)])
    src_pad = jnp.concatenate([src, jnp.zeros((pad_e,), jnp.int32)])
    dst_r = dst_pad.reshape(NW, E_CH, CH)
    src_r = src_pad.reshape(NW, E_CH, CH)
    dstg_r = jnp.minimum(dst_r, N - 1)
    qe, kve = _edge_gather(dstg_r, src_r, q, kv)

    Bb = 2048
    ge = EP // Bb
    erow = lambda w: pl.BlockSpec((Bb, w), lambda i: (i, 0))
    we0, we1, we2 = pl.pallas_call(
        _edge_dense_body,
        grid=(ge,),
        in_specs=[erow(256), erow(512), _full(0, (1, H))],
        out_specs=[erow(WE_W), erow(WE_W), erow(WE_W)],
        out_shape=[jax.ShapeDtypeStruct((EP, WE_W), f32)] * 3,
    )(qe, kve, khmax)

    zero_blk = jnp.zeros((ACC_PW, WE_W), f32)
    s0 = _edge_scatter(we0, dst_r, zero_blk)
    s1 = _edge_scatter(we1, dst_r, zero_blk)
    s2 = _edge_scatter(we2, dst_r, zero_blk)

    out = pl.pallas_call(
        _final_body,
        grid=(gn,),
        in_specs=[
            row(WE_W), row(WE_W), row(WE_W), row(WE_W), row(WE_W),
            row(WE_W), row(D),
            _full(0, (256, D)), _full(0, (1, D)),
            _full(0, (D, 32)), _full(0, (1, 32)),
            _full(0, (32, 1)), _full(0, (1, 1)),
        ],
        out_specs=pl.BlockSpec((BN, 1), lambda i: (i, 0)),
        out_shape=jax.ShapeDtypeStruct((N, 1), f32),
    )(s0[0, :N], s0[1, :N], s1[0, :N], s1[1, :N], s2[0, :N], s2[1, :N],
      comb, gnn_Wo, gnn_bo.reshape(1, -1),
      fin_W1, fin_b1.reshape(1, -1), fin_W2, fin_b2.reshape(1, -1))
    return out


# speedup vs baseline: 9.7061x; 1.0715x over previous
import functools

import jax
import jax.numpy as jnp
import numpy as np
from jax import lax
from jax.experimental import pallas as pl
from jax.experimental.pallas import tpu as pltpu
from jax.experimental.pallas import tpu_sc as plsc

N = 10000
E = 160000
T = 10
B = 1000
SKD = 75
D = 320
H = 8
DH = 32

NC = 2
NS = 16
NW = NC * NS
CH = 128

NT_CH = 3
NT = NW * NT_CH * CH
E_CH = 40
EP = NW * E_CH * CH
EPW = E_CH * CH
GCH = 64
G_CH = EPW // GCH
ACC_PW = 632
ACC_N = NS * ACC_PW
WE_W = 128
GROUPS = ((0, 1, 2), (3, 4, 5), (6, 7))

def _gelu(x):
    return 0.5 * x * (1.0 + lax.erf(x * np.float32(1.0 / np.sqrt(2.0))))


@functools.cache
def _sc_kernels():
    mesh = plsc.VectorSubcoreMesh(core_axis_name="c", subcore_axis_name="s",
                                  num_cores=NC, num_subcores=NS)

    @functools.partial(
        pl.kernel,
        out_type=jax.ShapeDtypeStruct((NT, 128), jnp.float32),
        mesh=mesh,
        scratch_types=[
            pltpu.VMEM((NT_CH, CH), jnp.int32),
            pltpu.VMEM((NT_CH * CH, 128), jnp.float32),
            pltpu.SemaphoreType.DMA,
        ],
    )
    def time_gather(time_hbm, temb_hbm, out_hbm, idx_v, rows_v, sem):
        wid = lax.axis_index("s") * NC + lax.axis_index("c")
        pltpu.sync_copy(time_hbm.at[wid], idx_v)

        def body(j, carry):
            pltpu.async_copy(temb_hbm.at[idx_v.at[j]],
                             rows_v.at[pl.ds(j * CH, CH)], sem).wait()
            return carry

        lax.fori_loop(0, NT_CH, body, 0)
        pltpu.sync_copy(rows_v,
                        out_hbm.at[pl.ds(wid * (NT_CH * CH), NT_CH * CH)])

    @functools.partial(
        pl.kernel,
        out_type=(jax.ShapeDtypeStruct((EP, 256), jnp.float32),
                  jax.ShapeDtypeStruct((EP, 512), jnp.float32)),
        mesh=mesh,
        scratch_types=[
            pltpu.VMEM((G_CH, GCH), jnp.int32),
            pltpu.VMEM((G_CH, GCH), jnp.int32),
            pltpu.VMEM((GCH, 256), jnp.float32),
            pltpu.VMEM((GCH, 256), jnp.float32),
            pltpu.VMEM((GCH, 512), jnp.float32),
            pltpu.VMEM((GCH, 512), jnp.float32),
            pltpu.SemaphoreType.DMA,
            pltpu.SemaphoreType.DMA,
            pltpu.SemaphoreType.DMA,
            pltpu.SemaphoreType.DMA,
        ],
    )
    def edge_gather(dst_hbm, src_hbm, q_hbm, kv_hbm, qe_hbm, kve_hbm,
                    idxd, idxs, qr0, qr1, kvr0, kvr1, sq0, sq1, skv0, skv1):
        wid = lax.axis_index("s") * NC + lax.axis_index("c")
        pltpu.sync_copy(dst_hbm.at[wid], idxd)
        pltpu.sync_copy(src_hbm.at[wid], idxs)
        bufs = ((qr0, kvr0, sq0, skv0), (qr1, kvr1, sq1, skv1))

        def fire(j, qr, kvr, sq, skv):
            pltpu.async_copy(q_hbm.at[idxd.at[j]], qr, sq)
            pltpu.async_copy(kv_hbm.at[idxs.at[j]], kvr, skv)

        def drain_write(j, qr, kvr, sq, skv):
            base = wid * EPW + j * GCH
            pltpu.make_async_copy(q_hbm.at[idxd.at[j]], qr, sq).wait()
            pltpu.make_async_copy(kv_hbm.at[idxs.at[j]], kvr, skv).wait()
            pltpu.sync_copy(qr, qe_hbm.at[pl.ds(base, GCH)])
            pltpu.sync_copy(kvr, kve_hbm.at[pl.ds(base, GCH)])

        fire(0, *bufs[0])

        def body(t, carry):
            g0 = 2 * t
            fire(g0 + 1, *bufs[1])
            drain_write(g0, *bufs[0])

            @pl.when(t + 1 < G_CH // 2)
            def _():
                fire(g0 + 2, *bufs[0])

            drain_write(g0 + 1, *bufs[1])
            return carry

        lax.fori_loop(0, G_CH // 2, body, 0)

    @functools.partial(
        pl.kernel,
        out_type=jax.ShapeDtypeStruct((NC, ACC_N, WE_W), jnp.float32),
        mesh=mesh,
        scratch_types=[
            pltpu.VMEM((E_CH, CH), jnp.int32),
            pltpu.VMEM((CH, WE_W), jnp.float32),
            pltpu.VMEM_SHARED((ACC_N, WE_W), jnp.float32),
        ],
    )
    def edge_scatter(we_hbm, dst_hbm, zero_hbm, out_hbm, idxd, rows_v, acc):
        cid = lax.axis_index("c")
        sid = lax.axis_index("s")
        wid = sid * NC + cid
        pltpu.sync_copy(zero_hbm, acc.at[pl.ds(sid * ACC_PW, ACC_PW)])
        plsc.subcore_barrier()
        pltpu.sync_copy(dst_hbm.at[wid], idxd)

        def body(j, carry):
            base = wid * EPW + j * CH
            pltpu.sync_copy(we_hbm.at[pl.ds(base, CH)], rows_v)
            pltpu.sync_copy(rows_v, acc.at[idxd.at[j]], add=True)
            return carry

        lax.fori_loop(0, E_CH, body, 0)
        plsc.subcore_barrier()
        pltpu.sync_copy(acc.at[pl.ds(sid * ACC_PW, ACC_PW)],
                        out_hbm.at[cid, pl.ds(sid * ACC_PW, ACC_PW)])

    return time_gather, edge_gather, edge_scatter


def _dense_pre_body(skel_ref, xy_ref, tf_ref, eW1, eb1, eW2, eb2,
                    pW1, pb1, pW2, pb2, mW1, mb1, mW2, mb2,
                    wq, wk, wv, comb_o, q_o, kv_o, kn_o):
    x = skel_ref[...]
    skel = _gelu(x @ eW1[...] + eb1[...]) @ eW2[...] + eb2[...]
    pf = _gelu(xy_ref[...] @ pW1[...] + pb1[...]) @ pW2[...] + pb2[...]
    cin = jnp.concatenate([skel, pf, tf_ref[...]], axis=1)
    comb = _gelu(cin @ mW1[...] + mb1[...]) @ mW2[...] + mb2[...]
    comb_o[...] = comb
    q = comb @ wq[...]
    k = comb @ wk[...]
    v = comb @ wv[...]
    q_o[...] = q
    kv_o[...] = jnp.concatenate([k, v], axis=1)
    ksq = k * k
    kn_o[...] = jnp.sqrt(jnp.concatenate(
        [jnp.sum(ksq[:, h * DH:(h + 1) * DH], axis=1, keepdims=True)
         for h in range(H)], axis=1))


def _edge_dense_body(qe_ref, kve_ref, km_ref, we0_o, we1_o, we2_o):
    qe = qe_ref[...]
    kve = kve_ref[...]
    inv = np.float32(1.0 / np.sqrt(DH))
    per_head = []
    for h in range(H):
        qh = qe[:, h * DH:(h + 1) * DH]
        kh = kve[:, h * DH:(h + 1) * DH]
        vh = kve[:, 256 + h * DH:256 + (h + 1) * DH]
        s = jnp.sum(qh * kh, axis=1, keepdims=True) * inv
        qn = jnp.sqrt(jnp.sum(qh * qh, axis=1, keepdims=True))
        ce = qn * km_ref[0:1, h:h + 1] * inv
        ex = jnp.exp(s - ce)
        per_head.append((ex * vh, ex))
    for hs, o_ref in zip(GROUPS, (we0_o, we1_o, we2_o)):
        wvs = [per_head[h][0] for h in hs]
        exs = [per_head[h][1] for h in hs]
        pad = WE_W - (DH + 1) * len(hs)
        z = jnp.zeros((qe.shape[0], pad), jnp.float32)
        o_ref[...] = jnp.concatenate(wvs + exs + [z], axis=1)


def _final_body(s0a, s0b, s1a, s1b, s2a, s2b, comb_ref, wo, bo,
                fW1, fb1, fW2, fb2, out_o):
    pieces = []
    for hs, (ga, gb) in zip(GROUPS, ((s0a, s0b), (s1a, s1b), (s2a, s2b))):
        g = ga[...] + gb[...]
        nh = len(hs)
        for i in range(nh):
            num = g[:, i * DH:(i + 1) * DH]
            den = g[:, nh * DH + i:nh * DH + i + 1]
            pieces.append(num / (den + 1e-16))
    agg = jnp.concatenate(pieces, axis=1)
    feats = agg @ wo[...] + bo[...]
    x = feats + comb_ref[...]
    hid = _gelu(x @ fW1[...] + fb1[...])
    out_o[...] = hid @ fW2[...] + fb2[...]


def _full(i, o):
    return pl.BlockSpec(o, lambda i_: tuple(0 for _ in o))


def kernel(xy_pos, time, skeletons, edge_index, batch, time_emb,
           enc_W1, enc_b1, enc_W2, enc_b2, pos_W1, pos_b1, pos_W2, pos_b2,
           mlp_W1, mlp_b1, mlp_W2, mlp_b2,
           gnn_Wq, gnn_Wk, gnn_Wv, gnn_Wo, gnn_bo,
           fin_W1, fin_b1, fin_W2, fin_b2):
    _time_gather, _edge_gather, _edge_scatter = _sc_kernels()
    f32 = jnp.float32
    time_i = time.astype(jnp.int32)
    src = edge_index[0].astype(jnp.int32)
    dst = edge_index[1].astype(jnp.int32)

    skel_in = jnp.transpose(skeletons, (1, 0, 2)).reshape(N, SKD)

    time_pad = jnp.concatenate(
        [time_i, jnp.zeros((NT - N,), jnp.int32)]).reshape(NW, NT_CH, CH)
    temb_pad = jnp.pad(time_emb, ((0, 0), (0, 128 - time_emb.shape[1])))
    tfeat = _time_gather(time_pad, temb_pad)[:N, :32]

    BN = 1000
    gn = N // BN
    row = lambda w: pl.BlockSpec((BN, w), lambda i: (i, 0))
    comb, q, kv, kn = pl.pallas_call(
        _dense_pre_body,
        grid=(gn,),
        in_specs=[
            row(SKD), row(1), row(32),
            _full(0, (SKD, 256)), _full(0, (1, 256)),
            _full(0, (256, 256)), _full(0, (1, 256)),
            _full(0, (1, 16)), _full(0, (1, 16)),
            _full(0, (16, 32)), _full(0, (1, 32)),
            _full(0, (D, 128)), _full(0, (1, 128)),
            _full(0, (128, D)), _full(0, (1, D)),
            _full(0, (D, 256)), _full(0, (D, 256)), _full(0, (D, 256)),
        ],
        out_specs=[row(D), row(256), row(512), row(H)],
        out_shape=[
            jax.ShapeDtypeStruct((N, D), f32),
            jax.ShapeDtypeStruct((N, 256), f32),
            jax.ShapeDtypeStruct((N, 512), f32),
            jax.ShapeDtypeStruct((N, H), f32),
        ],
    )(skel_in, xy_pos, tfeat,
      enc_W1, enc_b1.reshape(1, -1), enc_W2, enc_b2.reshape(1, -1),
      pos_W1, pos_b1.reshape(1, -1), pos_W2, pos_b2.reshape(1, -1),
      mlp_W1, mlp_b1.reshape(1, -1), mlp_W2, mlp_b2.reshape(1, -1),
      gnn_Wq, gnn_Wk, gnn_Wv)

    khmax = jnp.max(kn, axis=0).reshape(1, H)

    pad_e = EP - E
    dst_pad = jnp.concatenate([dst, jnp.full((pad_e,), N, jnp.int32)])
    src_pad = jnp.concatenate([src, jnp.zeros((pad_e,), jnp.int32)])
    dst_r = dst_pad.reshape(NW, E_CH, CH)
    dstg_r = jnp.minimum(dst_pad, N - 1).reshape(NW, G_CH, GCH)
    srcg_r = src_pad.reshape(NW, G_CH, GCH)
    qe, kve = _edge_gather(dstg_r, srcg_r, q, kv)

    Bb = 2048
    ge = EP // Bb
    erow = lambda w: pl.BlockSpec((Bb, w), lambda i: (i, 0))
    we0, we1, we2 = pl.pallas_call(
        _edge_dense_body,
        grid=(ge,),
        in_specs=[erow(256), erow(512), _full(0, (1, H))],
        out_specs=[erow(WE_W), erow(WE_W), erow(WE_W)],
        out_shape=[jax.ShapeDtypeStruct((EP, WE_W), f32)] * 3,
    )(qe, kve, khmax)

    zero_blk = jnp.zeros((ACC_PW, WE_W), f32)
    s0 = _edge_scatter(we0, dst_r, zero_blk)
    s1 = _edge_scatter(we1, dst_r, zero_blk)
    s2 = _edge_scatter(we2, dst_r, zero_blk)

    out = pl.pallas_call(
        _final_body,
        grid=(gn,),
        in_specs=[
            row(WE_W), row(WE_W), row(WE_W), row(WE_W), row(WE_W),
            row(WE_W), row(D),
            _full(0, (256, D)), _full(0, (1, D)),
            _full(0, (D, 32)), _full(0, (1, 32)),
            _full(0, (32, 1)), _full(0, (1, 1)),
        ],
        out_specs=pl.BlockSpec((BN, 1), lambda i: (i, 0)),
        out_shape=jax.ShapeDtypeStruct((N, 1), f32),
    )(s0[0, :N], s0[1, :N], s1[0, :N], s1[1, :N], s2[0, :N], s2[1, :N],
      comb, gnn_Wo, gnn_bo.reshape(1, -1),
      fin_W1, fin_b1.reshape(1, -1), fin_W2, fin_b2.reshape(1, -1))
    return out


# speedup vs baseline: 10.1614x; 1.0469x over previous
import functools

import jax
import jax.numpy as jnp
import numpy as np
from jax import lax
from jax.experimental import pallas as pl
from jax.experimental.pallas import tpu as pltpu
from jax.experimental.pallas import tpu_sc as plsc

N = 10000
E = 160000
T = 10
B = 1000
SKD = 75
D = 320
H = 8
DH = 32

NC = 2
NS = 16
NW = NC * NS
CH = 128

NT_CH = 3
NT = NW * NT_CH * CH
E_CH = 40
EP = NW * E_CH * CH
EPW = E_CH * CH
GCH = 64
G_CH = EPW // GCH
ACC_PW = 632
ACC_N = NS * ACC_PW
WE_W = 128
GROUPS = ((0, 1, 2), (3, 4, 5), (6, 7))

def _gelu(x):
    return 0.5 * x * (1.0 + lax.erf(x * np.float32(1.0 / np.sqrt(2.0))))


@functools.cache
def _sc_kernels():
    mesh = plsc.VectorSubcoreMesh(core_axis_name="c", subcore_axis_name="s",
                                  num_cores=NC, num_subcores=NS)

    @functools.partial(
        pl.kernel,
        out_type=jax.ShapeDtypeStruct((NT, 128), jnp.float32),
        mesh=mesh,
        scratch_types=[
            pltpu.VMEM((NT_CH, CH), jnp.int32),
            pltpu.VMEM((NT_CH * CH, 128), jnp.float32),
            pltpu.SemaphoreType.DMA,
        ],
    )
    def time_gather(time_hbm, temb_hbm, out_hbm, idx_v, rows_v, sem):
        wid = lax.axis_index("s") * NC + lax.axis_index("c")
        pltpu.sync_copy(time_hbm.at[wid], idx_v)
        for j in range(NT_CH):
            pltpu.async_copy(temb_hbm.at[idx_v.at[j]],
                             rows_v.at[pl.ds(j * CH, CH)], sem)
        for j in range(NT_CH):
            pltpu.make_async_copy(temb_hbm.at[idx_v.at[j]],
                                  rows_v.at[pl.ds(j * CH, CH)], sem).wait()
        pltpu.sync_copy(rows_v,
                        out_hbm.at[pl.ds(wid * (NT_CH * CH), NT_CH * CH)])

    @functools.partial(
        pl.kernel,
        out_type=(jax.ShapeDtypeStruct((EP, 256), jnp.float32),
                  jax.ShapeDtypeStruct((EP, 512), jnp.float32)),
        mesh=mesh,
        scratch_types=[
            pltpu.VMEM((G_CH, GCH), jnp.int32),
            pltpu.VMEM((G_CH, GCH), jnp.int32),
            pltpu.VMEM((GCH, 256), jnp.float32),
            pltpu.VMEM((GCH, 256), jnp.float32),
            pltpu.VMEM((GCH, 512), jnp.float32),
            pltpu.VMEM((GCH, 512), jnp.float32),
            pltpu.SemaphoreType.DMA,
            pltpu.SemaphoreType.DMA,
            pltpu.SemaphoreType.DMA,
            pltpu.SemaphoreType.DMA,
        ],
    )
    def edge_gather(dst_hbm, src_hbm, q_hbm, kv_hbm, qe_hbm, kve_hbm,
                    idxd, idxs, qr0, qr1, kvr0, kvr1, sq0, sq1, skv0, skv1):
        wid = lax.axis_index("s") * NC + lax.axis_index("c")
        pltpu.sync_copy(dst_hbm.at[wid], idxd)
        pltpu.sync_copy(src_hbm.at[wid], idxs)
        bufs = ((qr0, kvr0, sq0, skv0), (qr1, kvr1, sq1, skv1))

        def fire(j, qr, kvr, sq, skv):
            pltpu.async_copy(q_hbm.at[idxd.at[j]], qr, sq)
            pltpu.async_copy(kv_hbm.at[idxs.at[j]], kvr, skv)

        def drain_write(j, qr, kvr, sq, skv):
            base = wid * EPW + j * GCH
            pltpu.make_async_copy(q_hbm.at[idxd.at[j]], qr, sq).wait()
            pltpu.make_async_copy(kv_hbm.at[idxs.at[j]], kvr, skv).wait()
            pltpu.sync_copy(qr, qe_hbm.at[pl.ds(base, GCH)])
            pltpu.sync_copy(kvr, kve_hbm.at[pl.ds(base, GCH)])

        fire(0, *bufs[0])

        def body(t, carry):
            g0 = 2 * t
            fire(g0 + 1, *bufs[1])
            drain_write(g0, *bufs[0])

            @pl.when(t + 1 < G_CH // 2)
            def _():
                fire(g0 + 2, *bufs[0])

            drain_write(g0 + 1, *bufs[1])
            return carry

        lax.fori_loop(0, G_CH // 2, body, 0)

    @functools.partial(
        pl.kernel,
        out_type=jax.ShapeDtypeStruct((NC, ACC_N, WE_W), jnp.float32),
        mesh=mesh,
        scratch_types=[
            pltpu.VMEM((E_CH, CH), jnp.int32),
            pltpu.VMEM((CH, WE_W), jnp.float32),
            pltpu.VMEM((CH, WE_W), jnp.float32),
            pltpu.VMEM_SHARED((ACC_N, WE_W), jnp.float32),
            pltpu.SemaphoreType.DMA,
            pltpu.SemaphoreType.DMA,
        ],
    )
    def edge_scatter(we_hbm, dst_hbm, zero_hbm, out_hbm, idxd,
                     rw0, rw1, acc, sm0, sm1):
        cid = lax.axis_index("c")
        sid = lax.axis_index("s")
        wid = sid * NC + cid
        pltpu.sync_copy(zero_hbm, acc.at[pl.ds(sid * ACC_PW, ACC_PW)])
        plsc.subcore_barrier()
        pltpu.sync_copy(dst_hbm.at[wid], idxd)
        bufs = ((rw0, sm0), (rw1, sm1))

        def fire(j, rw, sm):
            base = wid * EPW + j * CH
            pltpu.async_copy(we_hbm.at[pl.ds(base, CH)], rw, sm)

        def drain_scatter(j, rw, sm):
            base = wid * EPW + j * CH
            pltpu.make_async_copy(we_hbm.at[pl.ds(base, CH)], rw, sm).wait()
            pltpu.sync_copy(rw, acc.at[idxd.at[j]], add=True)

        fire(0, *bufs[0])

        def body(t, carry):
            g0 = 2 * t
            fire(g0 + 1, *bufs[1])
            drain_scatter(g0, *bufs[0])

            @pl.when(t + 1 < E_CH // 2)
            def _():
                fire(g0 + 2, *bufs[0])

            drain_scatter(g0 + 1, *bufs[1])
            return carry

        lax.fori_loop(0, E_CH // 2, body, 0)
        plsc.subcore_barrier()
        pltpu.sync_copy(acc.at[pl.ds(sid * ACC_PW, ACC_PW)],
                        out_hbm.at[cid, pl.ds(sid * ACC_PW, ACC_PW)])

    return time_gather, edge_gather, edge_scatter


def _dense_pre_body(skel_ref, xy_ref, tf_ref, eW1, eb1, eW2, eb2,
                    pW1, pb1, pW2, pb2, mW1, mb1, mW2, mb2,
                    wq, wk, wv, comb_o, q_o, kv_o, kn_o):
    x = skel_ref[...]
    skel = _gelu(x @ eW1[...] + eb1[...]) @ eW2[...] + eb2[...]
    pf = _gelu(xy_ref[...] @ pW1[...] + pb1[...]) @ pW2[...] + pb2[...]
    cin = jnp.concatenate([skel, pf, tf_ref[...]], axis=1)
    comb = _gelu(cin @ mW1[...] + mb1[...]) @ mW2[...] + mb2[...]
    comb_o[...] = comb
    q = comb @ wq[...]
    k = comb @ wk[...]
    v = comb @ wv[...]
    q_o[...] = q
    kv_o[...] = jnp.concatenate([k, v], axis=1)
    ksq = k * k
    kn_o[...] = jnp.sqrt(jnp.concatenate(
        [jnp.sum(ksq[:, h * DH:(h + 1) * DH], axis=1, keepdims=True)
         for h in range(H)], axis=1))


def _edge_dense_body(qe_ref, kve_ref, km_ref, we0_o, we1_o, we2_o):
    qe = qe_ref[...]
    kve = kve_ref[...]
    inv = np.float32(1.0 / np.sqrt(DH))
    per_head = []
    for h in range(H):
        qh = qe[:, h * DH:(h + 1) * DH]
        kh = kve[:, h * DH:(h + 1) * DH]
        vh = kve[:, 256 + h * DH:256 + (h + 1) * DH]
        s = jnp.sum(qh * kh, axis=1, keepdims=True) * inv
        qn = jnp.sqrt(jnp.sum(qh * qh, axis=1, keepdims=True))
        ce = qn * km_ref[0:1, h:h + 1] * inv
        ex = jnp.exp(s - ce)
        per_head.append((ex * vh, ex))
    for hs, o_ref in zip(GROUPS, (we0_o, we1_o, we2_o)):
        wvs = [per_head[h][0] for h in hs]
        exs = [per_head[h][1] for h in hs]
        pad = WE_W - (DH + 1) * len(hs)
        z = jnp.zeros((qe.shape[0], pad), jnp.float32)
        o_ref[...] = jnp.concatenate(wvs + exs + [z], axis=1)


def _final_body(s0a, s0b, s1a, s1b, s2a, s2b, comb_ref, wo, bo,
                fW1, fb1, fW2, fb2, out_o):
    pieces = []
    for hs, (ga, gb) in zip(GROUPS, ((s0a, s0b), (s1a, s1b), (s2a, s2b))):
        g = ga[...] + gb[...]
        nh = len(hs)
        for i in range(nh):
            num = g[:, i * DH:(i + 1) * DH]
            den = g[:, nh * DH + i:nh * DH + i + 1]
            pieces.append(num / (den + 1e-16))
    agg = jnp.concatenate(pieces, axis=1)
    feats = agg @ wo[...] + bo[...]
    x = feats + comb_ref[...]
    hid = _gelu(x @ fW1[...] + fb1[...])
    out_o[...] = hid @ fW2[...] + fb2[...]


def _full(i, o):
    return pl.BlockSpec(o, lambda i_: tuple(0 for _ in o))


def kernel(xy_pos, time, skeletons, edge_index, batch, time_emb,
           enc_W1, enc_b1, enc_W2, enc_b2, pos_W1, pos_b1, pos_W2, pos_b2,
           mlp_W1, mlp_b1, mlp_W2, mlp_b2,
           gnn_Wq, gnn_Wk, gnn_Wv, gnn_Wo, gnn_bo,
           fin_W1, fin_b1, fin_W2, fin_b2):
    _time_gather, _edge_gather, _edge_scatter = _sc_kernels()
    f32 = jnp.float32
    time_i = time.astype(jnp.int32)
    src = edge_index[0].astype(jnp.int32)
    dst = edge_index[1].astype(jnp.int32)

    skel_in = jnp.transpose(skeletons, (1, 0, 2)).reshape(N, SKD)

    time_pad = jnp.concatenate(
        [time_i, jnp.zeros((NT - N,), jnp.int32)]).reshape(NW, NT_CH, CH)
    temb_pad = jnp.pad(time_emb, ((0, 0), (0, 128 - time_emb.shape[1])))
    tfeat = _time_gather(time_pad, temb_pad)[:N, :32]

    BN = 1000
    gn = N // BN
    row = lambda w: pl.BlockSpec((BN, w), lambda i: (i, 0))
    comb, q, kv, kn = pl.pallas_call(
        _dense_pre_body,
        grid=(gn,),
        in_specs=[
            row(SKD), row(1), row(32),
            _full(0, (SKD, 256)), _full(0, (1, 256)),
            _full(0, (256, 256)), _full(0, (1, 256)),
            _full(0, (1, 16)), _full(0, (1, 16)),
            _full(0, (16, 32)), _full(0, (1, 32)),
            _full(0, (D, 128)), _full(0, (1, 128)),
            _full(0, (128, D)), _full(0, (1, D)),
            _full(0, (D, 256)), _full(0, (D, 256)), _full(0, (D, 256)),
        ],
        out_specs=[row(D), row(256), row(512), row(H)],
        out_shape=[
            jax.ShapeDtypeStruct((N, D), f32),
            jax.ShapeDtypeStruct((N, 256), f32),
            jax.ShapeDtypeStruct((N, 512), f32),
            jax.ShapeDtypeStruct((N, H), f32),
        ],
    )(skel_in, xy_pos, tfeat,
      enc_W1, enc_b1.reshape(1, -1), enc_W2, enc_b2.reshape(1, -1),
      pos_W1, pos_b1.reshape(1, -1), pos_W2, pos_b2.reshape(1, -1),
      mlp_W1, mlp_b1.reshape(1, -1), mlp_W2, mlp_b2.reshape(1, -1),
      gnn_Wq, gnn_Wk, gnn_Wv)

    khmax = jnp.max(kn, axis=0).reshape(1, H)

    pad_e = EP - E
    dst_pad = jnp.concatenate([dst, jnp.full((pad_e,), N, jnp.int32)])
    src_pad = jnp.concatenate([src, jnp.zeros((pad_e,), jnp.int32)])
    dst_r = dst_pad.reshape(NW, E_CH, CH)
    dstg_r = jnp.minimum(dst_pad, N - 1).reshape(NW, G_CH, GCH)
    srcg_r = src_pad.reshape(NW, G_CH, GCH)
    qe, kve = _edge_gather(dstg_r, srcg_r, q, kv)

    Bb = 2048
    ge = EP // Bb
    erow = lambda w: pl.BlockSpec((Bb, w), lambda i: (i, 0))
    we0, we1, we2 = pl.pallas_call(
        _edge_dense_body,
        grid=(ge,),
        in_specs=[erow(256), erow(512), _full(0, (1, H))],
        out_specs=[erow(WE_W), erow(WE_W), erow(WE_W)],
        out_shape=[jax.ShapeDtypeStruct((EP, WE_W), f32)] * 3,
    )(qe, kve, khmax)

    zero_blk = jnp.zeros((ACC_PW, WE_W), f32)
    s0 = _edge_scatter(we0, dst_r, zero_blk)
    s1 = _edge_scatter(we1, dst_r, zero_blk)
    s2 = _edge_scatter(we2, dst_r, zero_blk)

    out = pl.pallas_call(
        _final_body,
        grid=(gn,),
        in_specs=[
            row(WE_W), row(WE_W), row(WE_W), row(WE_W), row(WE_W),
            row(WE_W), row(D),
            _full(0, (256, D)), _full(0, (1, D)),
            _full(0, (D, 32)), _full(0, (1, 32)),
            _full(0, (32, 1)), _full(0, (1, 1)),
        ],
        out_specs=pl.BlockSpec((BN, 1), lambda i: (i, 0)),
        out_shape=jax.ShapeDtypeStruct((N, 1), f32),
    )(s0[0, :N], s0[1, :N], s1[0, :N], s1[1, :N], s2[0, :N], s2[1, :N],
      comb, gnn_Wo, gnn_bo.reshape(1, -1),
      fin_W1, fin_b1.reshape(1, -1), fin_W2, fin_b2.reshape(1, -1))
    return out
